# baseline (speedup 1.0000x reference)
import functools

import jax
import jax.numpy as jnp
from jax import lax
from jax.experimental import pallas as pl
from jax.experimental.pallas import tpu as pltpu
from jax.experimental.pallas import tpu_sc as plsc

_B = 16384
_D = 16
_W = 7
_ROW_SHIFT = 1
_NW = 32
_BPW = _B // _NW
_CH = _BPW // 16
_NROWS = 1000009
_PITCH = 1000016
_WORDS = _BPW * _W * _D
_GC = 128
_NG = _WORDS // _GC
_CBASE = [c * _PITCH for c in range(_D)]


def _build_gather():
    mesh = plsc.VectorSubcoreMesh(core_axis_name="c", subcore_axis_name="s")

    @functools.partial(
        pl.kernel,
        mesh=mesh,
        compiler_params=pltpu.CompilerParams(
            use_tc_tiling_on_sc=False, needs_layout_passes=False
        ),
        out_type=jax.ShapeDtypeStruct((_B * _W * _D,), jnp.float32),
        scratch_types=[
            pltpu.VMEM((_BPW,), jnp.int32),
            pltpu.VMEM((_WORDS,), jnp.int32),
            pltpu.VMEM((_WORDS,), jnp.float32),
            pltpu.SemaphoreType.DMA,
        ],
    )
    def gather_kernel(idx_hbm, embt_hbm, out_hbm, idx_v, exp_v, rows_v, sem):
        wid = lax.axis_index("s") * 2 + lax.axis_index("c")
        base = wid * _BPW
        pltpu.sync_copy(idx_hbm.at[pl.ds(base, _BPW)], idx_v)

        pos0 = lax.iota(jnp.int32, 16) * (_W * _D)
        emb_row = embt_hbm.at[0]

        def expand(c, carry):
            r1 = idx_v[pl.ds(c * 16, 16)] + _ROW_SHIFT
            pb = pos0 + c * (16 * _W * _D)
            for j in range(_W):
                t = r1 + j
                for d in range(_D):
                    plsc.store_scatter(
                        exp_v, [pb + (j * _D + d)], t + _CBASE[d]
                    )
            return carry

        lax.fori_loop(0, _CH, expand, 0)

        def fire(g, carry):
            pltpu.async_copy(
                emb_row.at[exp_v.at[pl.ds(g * _GC, _GC)]],
                rows_v.at[pl.ds(g * _GC, _GC)],
                sem,
            )
            return carry

        lax.fori_loop(0, _NG, fire, 0)
        pltpu.make_async_copy(emb_row.at[pl.ds(0, _WORDS)], rows_v, sem).wait()
        pltpu.sync_copy(rows_v, out_hbm.at[pl.ds(base * _W * _D, _WORDS)])

    return gather_kernel


def kernel(inputs, embedding):
    flat = _build_gather()(inputs.astype(jnp.int32), embedding.T)
    return flat.reshape(_B, _W, _D)
